# baseline (speedup 1.0000x reference)
import functools

import jax
import jax.numpy as jnp
from jax.experimental import pallas as pl
from jax.experimental.pallas import tpu as pltpu

S = 8
K = 2
D = 768
H = 12
DH = 64
DFF = 768
EPS = 1e-6
NEG = -1e9

B = 4
T = 2048
NPAIR = B * K

BLK_T = 1024
BLK_Q = 512
BLK_K = 512


def _rot_half(x):
    half = x.shape[-1] // 2
    return jnp.concatenate([-x[:, half:], x[:, :half]], axis=-1)


def _rms_norm(x, w):
    return x * jax.lax.rsqrt(jnp.mean(x * x, axis=-1, keepdims=True) + EPS) * w


def _dot(a, b):
    return jax.lax.dot_general(a, b, (((1,), (0,)), ((), ())),
                               preferred_element_type=jnp.float32)


def _dot_t(a, b):
    return jax.lax.dot_general(a, b, (((1,), (1,)), ((), ())),
                               preferred_element_type=jnp.float32)


def _mm(a, b):
    return jax.lax.dot_general(a.astype(jnp.bfloat16), b.astype(jnp.bfloat16),
                               (((1,), (0,)), ((), ())),
                               preferred_element_type=jnp.float32)


def _mm_t(a, b):
    return jax.lax.dot_general(a.astype(jnp.bfloat16), b.astype(jnp.bfloat16),
                               (((1,), (1,)), ((), ())),
                               preferred_element_type=jnp.float32)


def _router_kernel(hs_ref, wr_ref, temp_ref, logits_ref, idx_ref, w_ref):
    b = pl.program_id(0)
    x = hs_ref[0]
    mean = jnp.mean(x, axis=0, keepdims=True)
    lg = _dot_t(mean, wr_ref[...])
    t = jnp.clip(temp_ref[0, 0], 0.1, 10.0)
    logits_ref[pl.ds(b, 1), :] = lg / t

    @pl.when(b == B - 1)
    def _finish():
        l = logits_ref[...]
        iota = jax.lax.broadcasted_iota(jnp.int32, (B, S), 1)
        m1 = jnp.max(l, axis=1, keepdims=True)
        i1 = jnp.min(jnp.where(l == m1, iota, S), axis=1, keepdims=True)
        l2 = jnp.where(iota == i1, NEG, l)
        m2 = jnp.max(l2, axis=1, keepdims=True)
        i2 = jnp.min(jnp.where(l2 == m2, iota, S), axis=1, keepdims=True)
        idx_ref[...] = jnp.concatenate([i1, i2], axis=1)
        ed = jnp.exp(m2 - m1)
        w1 = 1.0 / (1.0 + ed)
        w_ref[...] = jnp.concatenate([w1, ed * w1], axis=1)


def _router(hidden_states, Wr, temperature):
    return pl.pallas_call(
        _router_kernel,
        grid=(B,),
        in_specs=[
            pl.BlockSpec((1, T, D), lambda b: (b, 0, 0)),
            pl.BlockSpec((S, D), lambda b: (0, 0)),
            pl.BlockSpec((1, 1), lambda b: (0, 0)),
        ],
        out_specs=[
            pl.BlockSpec((B, S), lambda b: (0, 0)),
            pl.BlockSpec((B, K), lambda b: (0, 0)),
            pl.BlockSpec((B, K), lambda b: (0, 0)),
        ],
        out_shape=[
            jax.ShapeDtypeStruct((B, S), jnp.float32),
            jax.ShapeDtypeStruct((B, K), jnp.int32),
            jax.ShapeDtypeStruct((B, K), jnp.float32),
        ],
    )(hidden_states, Wr, temperature.reshape(1, 1))


def _qkv_kernel(idx_ref, x_ref, ln1_ref, wq_ref, wk_ref, wv_ref,
                cos_ref, sin_ref, q_ref, k_ref, v_ref):
    h = _rms_norm(x_ref[0], ln1_ref[0, 0])
    cos, sin = cos_ref[...], sin_ref[...]
    half = DH // 2
    fh = (jax.lax.broadcasted_iota(jnp.int32, (1, D), 1) % DH) < half

    def rope(z):
        zm = jnp.concatenate([z[:, half:], z[:, :half]], axis=1)
        zp = jnp.concatenate([z[:, -half:], z[:, :-half]], axis=1)
        return z * cos + jnp.where(fh, -zm, zp) * sin

    q_ref[0] = rope(_dot(h, wq_ref[0]))
    k_ref[0] = rope(_dot(h, wk_ref[0]))
    v_ref[0] = _dot(h, wv_ref[0])


def _qkv(hidden_states, ln1r, Wq, Wk, Wv, cos_t, sin_t, flat_idx):
    nt = T // BLK_T
    grid_spec = pltpu.PrefetchScalarGridSpec(
        num_scalar_prefetch=1,
        grid=(NPAIR, nt),
        in_specs=[
            pl.BlockSpec((1, BLK_T, D), lambda e, t, idx: (e // K, t, 0)),
            pl.BlockSpec((1, 1, D), lambda e, t, idx: (idx[e], 0, 0)),
            pl.BlockSpec((1, D, D), lambda e, t, idx: (idx[e], 0, 0)),
            pl.BlockSpec((1, D, D), lambda e, t, idx: (idx[e], 0, 0)),
            pl.BlockSpec((1, D, D), lambda e, t, idx: (idx[e], 0, 0)),
            pl.BlockSpec((BLK_T, D), lambda e, t, idx: (t, 0)),
            pl.BlockSpec((BLK_T, D), lambda e, t, idx: (t, 0)),
        ],
        out_specs=[
            pl.BlockSpec((1, BLK_T, D), lambda e, t, idx: (e, t, 0)),
            pl.BlockSpec((1, BLK_T, D), lambda e, t, idx: (e, t, 0)),
            pl.BlockSpec((1, BLK_T, D), lambda e, t, idx: (e, t, 0)),
        ],
    )
    out_shape = [jax.ShapeDtypeStruct((NPAIR, T, D), jnp.float32)] * 3
    return pl.pallas_call(_qkv_kernel, grid_spec=grid_spec,
                          compiler_params=pltpu.CompilerParams(
                              dimension_semantics=("parallel", "parallel")),
                          out_shape=out_shape)(
        flat_idx, hidden_states, ln1r, Wq, Wk, Wv, cos_t, sin_t)


def _attn_kernel(q_ref, k_ref, v_ref, out_ref):
    qb = pl.program_id(1)
    scale = 1.0 / (DH ** 0.5)
    q_all = q_ref[0]
    rowio = jax.lax.broadcasted_iota(jnp.int32, (BLK_Q, BLK_K), 0)
    colio = jax.lax.broadcasted_iota(jnp.int32, (BLK_Q, BLK_K), 1)

    for h in range(H):
        sl = slice(h * DH, (h + 1) * DH)
        qh = q_all[:, sl]

        def body(kb, carry):
            m, l, acc = carry
            kd = pl.ds(kb * BLK_K, BLK_K)
            s = _dot_t(qh, k_ref[0, kd, sl]) * scale
            m_new = jnp.maximum(m, jnp.max(s, axis=1, keepdims=True))
            p = jnp.exp(s - m_new)
            alpha = jnp.exp(m - m_new)
            l_new = l * alpha + jnp.sum(p, axis=1, keepdims=True)
            acc_new = acc * alpha + _dot(p, v_ref[0, kd, sl])
            return m_new, l_new, acc_new

        m0 = jnp.full((BLK_Q, 1), -1e30, jnp.float32)
        l0 = jnp.zeros((BLK_Q, 1), jnp.float32)
        a0 = jnp.zeros((BLK_Q, DH), jnp.float32)
        m, l, acc = jax.lax.fori_loop(0, qb, body, (m0, l0, a0))

        kd = pl.ds(qb * BLK_K, BLK_K)
        s = _dot_t(qh, k_ref[0, kd, sl]) * scale
        s = jnp.where(colio <= rowio, s, NEG)
        m_new = jnp.maximum(m, jnp.max(s, axis=1, keepdims=True))
        p = jnp.exp(s - m_new)
        alpha = jnp.exp(m - m_new)
        l = l * alpha + jnp.sum(p, axis=1, keepdims=True)
        acc = acc * alpha + _dot(p, v_ref[0, kd, sl])
        out_ref[0, :, sl] = acc / l


def _attn(q, k, v):
    nq = T // BLK_Q
    return pl.pallas_call(
        _attn_kernel,
        grid=(NPAIR, nq),
        in_specs=[
            pl.BlockSpec((1, BLK_Q, D), lambda e, qb: (e, qb, 0)),
            pl.BlockSpec((1, T, D), lambda e, qb: (e, 0, 0)),
            pl.BlockSpec((1, T, D), lambda e, qb: (e, 0, 0)),
        ],
        out_specs=pl.BlockSpec((1, BLK_Q, D), lambda e, qb: (e, qb, 0)),
        out_shape=jax.ShapeDtypeStruct((NPAIR, T, D), jnp.float32),
        compiler_params=pltpu.CompilerParams(
            dimension_semantics=("parallel", "parallel")),
    )(q, k, v)


def _ffn_kernel(idx_ref, w_ref, ctx_ref, x_ref, wo_ref, ln2_ref,
                wg_ref, wu_ref, wd_ref, out_ref):
    b = pl.program_id(0)
    k = pl.program_id(2)
    w = w_ref[K * b + k]
    x1 = x_ref[0] + _dot(ctx_ref[0], wo_ref[0])
    h2 = _rms_norm(x1, ln2_ref[0, 0])
    g = _dot(h2, wg_ref[0])
    u = _dot(h2, wu_ref[0])
    y = (x1 + _dot(jax.nn.silu(g) * u, wd_ref[0])) * w

    @pl.when(k == 0)
    def _init():
        out_ref[0] = y

    @pl.when(k != 0)
    def _acc():
        out_ref[0] = out_ref[0] + y


def _ffn_combine(ctx, hidden_states, Wo, ln2r, Wg, Wu, Wd, flat_idx, flat_w):
    nt = T // BLK_T
    grid_spec = pltpu.PrefetchScalarGridSpec(
        num_scalar_prefetch=2,
        grid=(B, nt, K),
        in_specs=[
            pl.BlockSpec((1, BLK_T, D), lambda b, t, k, idx, w: (K * b + k, t, 0)),
            pl.BlockSpec((1, BLK_T, D), lambda b, t, k, idx, w: (b, t, 0)),
            pl.BlockSpec((1, D, D), lambda b, t, k, idx, w: (idx[K * b + k], 0, 0)),
            pl.BlockSpec((1, 1, D), lambda b, t, k, idx, w: (idx[K * b + k], 0, 0)),
            pl.BlockSpec((1, D, DFF), lambda b, t, k, idx, w: (idx[K * b + k], 0, 0)),
            pl.BlockSpec((1, D, DFF), lambda b, t, k, idx, w: (idx[K * b + k], 0, 0)),
            pl.BlockSpec((1, DFF, D), lambda b, t, k, idx, w: (idx[K * b + k], 0, 0)),
        ],
        out_specs=pl.BlockSpec((1, BLK_T, D), lambda b, t, k, idx, w: (b, t, 0)),
    )
    return pl.pallas_call(
        _ffn_kernel, grid_spec=grid_spec,
        compiler_params=pltpu.CompilerParams(
            dimension_semantics=("parallel", "parallel", "arbitrary")),
        out_shape=jax.ShapeDtypeStruct((B, T, D), jnp.float32),
    )(flat_idx, flat_w, ctx, hidden_states, Wo, ln2r, Wg, Wu, Wd)


def kernel(cos, sin, hidden_states, temperature, Wr, ln1, Wq, Wk, Wv, Wo,
           ln2, Wg, Wu, Wd):
    logits, topk_idx, topk_w = _router(hidden_states, Wr, temperature)
    flat_idx = topk_idx.reshape(-1)
    flat_w = topk_w.reshape(-1)
    ln1r = ln1.reshape(S, 1, D)
    ln2r = ln2.reshape(S, 1, D)

    cos_t = jnp.tile(cos, (1, H))
    sin_t = jnp.tile(sin, (1, H))

    q, k, v = _qkv(hidden_states, ln1r, Wq, Wk, Wv, cos_t, sin_t, flat_idx)
    ctx = _attn(q, k, v)
    result = _ffn_combine(ctx, hidden_states, Wo, ln2r, Wg, Wu, Wd,
                          flat_idx, flat_w)
    return result, logits


# speedup vs baseline: 1.2258x; 1.2258x over previous
import functools

import jax
import jax.numpy as jnp
from jax.experimental import pallas as pl
from jax.experimental.pallas import tpu as pltpu
from jax.experimental.pallas import tpu_sc as plsc

S = 8
K = 2
D = 768
H = 12
DH = 64
DFF = 768
EPS = 1e-6
NEG = -1e9

B = 4
T = 2048
NPAIR = B * K

BLK_T = 1024
BLK_Q = 512
BLK_K = 512


def _rms_norm(x, w):
    return x * jax.lax.rsqrt(jnp.mean(x * x, axis=-1, keepdims=True) + EPS) * w


def _dot(a, b):
    return jax.lax.dot_general(a, b, (((1,), (0,)), ((), ())),
                               preferred_element_type=jnp.float32)


def _dot_t(a, b):
    return jax.lax.dot_general(a, b, (((1,), (1,)), ((), ())),
                               preferred_element_type=jnp.float32)


def _router_kernel(hs_ref, wr_ref, temp_ref, logits_ref):
    b = pl.program_id(0)
    x = hs_ref[0]
    mean = jnp.mean(x, axis=0, keepdims=True)
    lg = _dot_t(mean, wr_ref[...])
    t = jnp.clip(temp_ref[0, 0], 0.1, 10.0)
    logits_ref[pl.ds(b, 1), :] = lg / t


def _router(hidden_states, Wr, temperature):
    return pl.pallas_call(
        _router_kernel,
        grid=(B,),
        in_specs=[
            pl.BlockSpec((1, T, D), lambda b: (b, 0, 0)),
            pl.BlockSpec((S, D), lambda b: (0, 0)),
            pl.BlockSpec((1, 1), lambda b: (0, 0)),
        ],
        out_specs=pl.BlockSpec((B, S), lambda b: (0, 0)),
        out_shape=jax.ShapeDtypeStruct((B, S), jnp.float32),
    )(hidden_states, Wr, temperature.reshape(1, 1))


def _sc_topk_kernel(logits_hbm, idx_hbm, w_hbm, vm, idx_scr, w_scr):
    cid = jax.lax.axis_index("c")
    sid = jax.lax.axis_index("s")

    @pl.when(jnp.logical_and(cid == 0, sid == 0))
    def _run():
        pltpu.sync_copy(logits_hbm, vm.at[pl.ds(0, B * S)])
        lane = jax.lax.iota(jnp.int32, 16)
        neg = jnp.full((16,), -1e30, jnp.float32)
        idx_vec = jnp.zeros((16,), jnp.int32)
        w_vec = jnp.zeros((16,), jnp.float32)
        for b in range(B):
            v = vm[pl.ds(S * b, 16)]
            v = jnp.where(lane < S, v, neg)
            m1 = jnp.max(v)
            i1 = jnp.min(jnp.where(v == m1, lane, 16))
            v2 = jnp.where(lane == i1, neg, v)
            m2 = jnp.max(v2)
            i2 = jnp.min(jnp.where(v2 == m2, lane, 16))
            ed = jnp.exp(jnp.full((16,), m2 - m1, jnp.float32))
            w1 = 1.0 / (1.0 + ed)
            w2 = ed * w1
            idx_vec = jnp.where(lane == K * b, jnp.full((16,), i1, jnp.int32),
                                idx_vec)
            idx_vec = jnp.where(lane == K * b + 1,
                                jnp.full((16,), i2, jnp.int32), idx_vec)
            w_vec = jnp.where(lane == K * b, w1, w_vec)
            w_vec = jnp.where(lane == K * b + 1, w2, w_vec)
        idx_scr[...] = idx_vec
        w_scr[...] = w_vec
        pltpu.sync_copy(idx_scr.at[pl.ds(0, B * K)], idx_hbm)
        pltpu.sync_copy(w_scr.at[pl.ds(0, B * K)], w_hbm)


def _sc_topk(logits_flat):
    fn = functools.partial(
        pl.kernel,
        mesh=plsc.VectorSubcoreMesh(core_axis_name="c", subcore_axis_name="s"),
        compiler_params=pltpu.CompilerParams(needs_layout_passes=False),
        out_type=[
            jax.ShapeDtypeStruct((B * K,), jnp.int32),
            jax.ShapeDtypeStruct((B * K,), jnp.float32),
        ],
        scratch_types=[
            pltpu.VMEM((5 * S,), jnp.float32),
            pltpu.VMEM((16,), jnp.int32),
            pltpu.VMEM((16,), jnp.float32),
        ],
    )(_sc_topk_kernel)
    return fn(logits_flat)


def _qkv_kernel(idx_ref, x_ref, ln1_ref, wq_ref, wk_ref, wv_ref,
                cos_ref, sin_ref, q_ref, k_ref, v_ref):
    h = _rms_norm(x_ref[0], ln1_ref[0, 0])
    cos, sin = cos_ref[...], sin_ref[...]
    half = DH // 2
    fh = (jax.lax.broadcasted_iota(jnp.int32, (1, D), 1) % DH) < half

    def rope(z):
        zm = jnp.concatenate([z[:, half:], z[:, :half]], axis=1)
        zp = jnp.concatenate([z[:, -half:], z[:, :-half]], axis=1)
        return z * cos + jnp.where(fh, -zm, zp) * sin

    q_ref[0] = (rope(_dot(h, wq_ref[0])) * (1.0 / DH ** 0.5)).astype(
        jnp.bfloat16)
    k_ref[0] = rope(_dot(h, wk_ref[0])).astype(jnp.bfloat16)
    v_ref[0] = _dot(h, wv_ref[0])


def _qkv(hidden_states, ln1r, Wq, Wk, Wv, cos_t, sin_t, flat_idx):
    nt = T // BLK_T
    grid_spec = pltpu.PrefetchScalarGridSpec(
        num_scalar_prefetch=1,
        grid=(NPAIR, nt),
        in_specs=[
            pl.BlockSpec((1, BLK_T, D), lambda e, t, idx: (e // K, t, 0)),
            pl.BlockSpec((1, 1, D), lambda e, t, idx: (idx[e], 0, 0)),
            pl.BlockSpec((1, D, D), lambda e, t, idx: (idx[e], 0, 0)),
            pl.BlockSpec((1, D, D), lambda e, t, idx: (idx[e], 0, 0)),
            pl.BlockSpec((1, D, D), lambda e, t, idx: (idx[e], 0, 0)),
            pl.BlockSpec((BLK_T, D), lambda e, t, idx: (t, 0)),
            pl.BlockSpec((BLK_T, D), lambda e, t, idx: (t, 0)),
        ],
        out_specs=[
            pl.BlockSpec((1, BLK_T, D), lambda e, t, idx: (e, t, 0)),
            pl.BlockSpec((1, BLK_T, D), lambda e, t, idx: (e, t, 0)),
            pl.BlockSpec((1, BLK_T, D), lambda e, t, idx: (e, t, 0)),
        ],
    )
    out_shape = [jax.ShapeDtypeStruct((NPAIR, T, D), jnp.bfloat16),
                 jax.ShapeDtypeStruct((NPAIR, T, D), jnp.bfloat16),
                 jax.ShapeDtypeStruct((NPAIR, T, D), jnp.float32)]
    return pl.pallas_call(_qkv_kernel, grid_spec=grid_spec,
                          compiler_params=pltpu.CompilerParams(
                              dimension_semantics=("parallel", "parallel")),
                          out_shape=out_shape)(
        flat_idx, hidden_states, ln1r, Wq, Wk, Wv, cos_t, sin_t)


GH = 4
NG = H // GH
GW = GH * DH


def _attn_kernel(q_ref, k_ref, v_ref, out_ref):
    qb = pl.program_id(1)
    q_all = q_ref[0]
    ones_dh = jnp.ones((1, DH), jnp.float32)
    rowio = jax.lax.broadcasted_iota(jnp.int32, (BLK_Q, GH * BLK_K), 0)
    colio = jax.lax.broadcasted_iota(jnp.int32, (BLK_Q, GH * BLK_K), 1) % BLK_K
    dmask = colio <= rowio
    bandio = jax.lax.broadcasted_iota(jnp.int32, (1, GW), 1) // DH
    bmask = [(bandio == i).astype(jnp.bfloat16) for i in range(GH)]
    vmask = [(bandio == i).astype(jnp.float32) for i in range(GH)]

    for g in range(NG):
        gsl = slice(g * GW, (g + 1) * GW)
        a = q_all[:, gsl]

        def step(kb, carry, masked):
            m, l, acc = carry
            kd = pl.ds(kb * BLK_K, BLK_K)
            kc = k_ref[0, kd, gsl]
            vc = v_ref[0, kd, gsl]
            bp = jnp.concatenate([kc * bmask[i] for i in range(GH)], axis=0)
            vp = jnp.concatenate([vc * vmask[i] for i in range(GH)], axis=0)
            s = _dot_t(a, bp)
            if masked:
                s = jnp.where(dmask, s, NEG)
            m_new, l_new, ps, als = [], [], [], []
            for i in range(GH):
                sb = s[:, i * BLK_K:(i + 1) * BLK_K]
                mi = jnp.maximum(m[i], jnp.max(sb, axis=1, keepdims=True))
                pi = jnp.exp(sb - mi)
                ai = jnp.exp(m[i] - mi)
                m_new.append(mi)
                l_new.append(l[i] * ai + jnp.sum(pi, axis=1, keepdims=True))
                ps.append(pi)
                als.append(ai)
            p_full = jnp.concatenate(ps, axis=1)
            alpha_full = jnp.concatenate([ai * ones_dh for ai in als], axis=1)
            acc_new = acc * alpha_full + _dot(p_full, vp)
            return tuple(m_new), tuple(l_new), acc_new

        m0 = tuple(jnp.full((BLK_Q, 1), -1e30, jnp.float32) for _ in range(GH))
        l0 = tuple(jnp.zeros((BLK_Q, 1), jnp.float32) for _ in range(GH))
        a0 = jnp.zeros((BLK_Q, GW), jnp.float32)
        carry = jax.lax.fori_loop(0, qb, lambda kb, c: step(kb, c, False),
                                  (m0, l0, a0))
        m, l, acc = step(qb, carry, True)
        linv = jnp.concatenate([(1.0 / li) * ones_dh for li in l], axis=1)
        out_ref[0, :, gsl] = acc * linv


def _attn(q, k, v):
    nq = T // BLK_Q
    return pl.pallas_call(
        _attn_kernel,
        grid=(NPAIR, nq),
        in_specs=[
            pl.BlockSpec((1, BLK_Q, D), lambda e, qb: (e, qb, 0)),
            pl.BlockSpec((1, T, D), lambda e, qb: (e, 0, 0)),
            pl.BlockSpec((1, T, D), lambda e, qb: (e, 0, 0)),
        ],
        out_specs=pl.BlockSpec((1, BLK_Q, D), lambda e, qb: (e, qb, 0)),
        out_shape=jax.ShapeDtypeStruct((NPAIR, T, D), jnp.float32),
        compiler_params=pltpu.CompilerParams(
            dimension_semantics=("parallel", "parallel")),
    )(q, k, v)


def _ffn_kernel(idx_ref, w_ref, ctx_ref, x_ref, wo_ref, ln2_ref,
                wg_ref, wu_ref, wd_ref, out_ref):
    b = pl.program_id(0)
    k = pl.program_id(2)
    w = w_ref[K * b + k]
    x1 = x_ref[0] + _dot(ctx_ref[0], wo_ref[0])
    h2 = _rms_norm(x1, ln2_ref[0, 0])
    g = _dot(h2, wg_ref[0])
    u = _dot(h2, wu_ref[0])
    y = (x1 + _dot(jax.nn.silu(g) * u, wd_ref[0])) * w

    @pl.when(k == 0)
    def _init():
        out_ref[0] = y

    @pl.when(k != 0)
    def _acc():
        out_ref[0] = out_ref[0] + y


def _ffn_combine(ctx, hidden_states, Wo, ln2r, Wg, Wu, Wd, flat_idx, flat_w):
    nt = T // BLK_T
    grid_spec = pltpu.PrefetchScalarGridSpec(
        num_scalar_prefetch=2,
        grid=(B, nt, K),
        in_specs=[
            pl.BlockSpec((1, BLK_T, D), lambda b, t, k, idx, w: (K * b + k, t, 0)),
            pl.BlockSpec((1, BLK_T, D), lambda b, t, k, idx, w: (b, t, 0)),
            pl.BlockSpec((1, D, D), lambda b, t, k, idx, w: (idx[K * b + k], 0, 0)),
            pl.BlockSpec((1, 1, D), lambda b, t, k, idx, w: (idx[K * b + k], 0, 0)),
            pl.BlockSpec((1, D, DFF), lambda b, t, k, idx, w: (idx[K * b + k], 0, 0)),
            pl.BlockSpec((1, D, DFF), lambda b, t, k, idx, w: (idx[K * b + k], 0, 0)),
            pl.BlockSpec((1, DFF, D), lambda b, t, k, idx, w: (idx[K * b + k], 0, 0)),
        ],
        out_specs=pl.BlockSpec((1, BLK_T, D), lambda b, t, k, idx, w: (b, t, 0)),
    )
    return pl.pallas_call(
        _ffn_kernel, grid_spec=grid_spec,
        compiler_params=pltpu.CompilerParams(
            dimension_semantics=("parallel", "parallel", "arbitrary")),
        out_shape=jax.ShapeDtypeStruct((B, T, D), jnp.float32),
    )(flat_idx, flat_w, ctx, hidden_states, Wo, ln2r, Wg, Wu, Wd)


def kernel(cos, sin, hidden_states, temperature, Wr, ln1, Wq, Wk, Wv, Wo,
           ln2, Wg, Wu, Wd):
    logits = _router(hidden_states, Wr, temperature)
    flat_idx, flat_w = _sc_topk(logits.reshape(-1))
    ln1r = ln1.reshape(S, 1, D)
    ln2r = ln2.reshape(S, 1, D)

    cos_t = jnp.tile(cos, (1, H))
    sin_t = jnp.tile(sin, (1, H))

    q, k, v = _qkv(hidden_states, ln1r, Wq, Wk, Wv, cos_t, sin_t, flat_idx)
    ctx = _attn(q, k, v)
    result = _ffn_combine(ctx, hidden_states, Wo, ln2r, Wg, Wu, Wd,
                          flat_idx, flat_w)
    return result, logits
